# baseline (speedup 1.0000x reference)
import math

import jax
import jax.numpy as jnp
from jax import lax
from jax.experimental import pallas as pl
from jax.experimental.pallas import tpu as pltpu
from jax.experimental.pallas import tpu_sc as plsc

N = 2048
H = 768
F = 3072
E = 8
EPS = 1e-12
T = 256
NB = 24
NPAD = NB * T
FB = 768
NFB = F // FB
NC, NS = 2, 16
NW = NC * NS
TOK_SC = N // NS
ROW_W = NPAD // NW
CH = 64
NCH = ROW_W // CH
TOK_W = N // NW


def _router_body(x_ref, rw_ref, rb_ref,
                 inv0_ref, inv1_ref, w0_ref, w1_ref, be_ref, na_ref):
    x = x_ref[...]
    lg = lax.dot_general(x, rw_ref[...], (((1,), (1,)), ((), ())),
                         preferred_element_type=jnp.float32) + rb_ref[...]
    iota = lax.broadcasted_iota(jnp.int32, (N, E), 1)
    v0 = jnp.max(lg, axis=1, keepdims=True)
    i0 = jnp.min(jnp.where(lg == v0, iota, E), axis=1, keepdims=True)
    m0 = iota == i0
    lgm = jnp.where(m0, -jnp.inf, lg)
    v1 = jnp.max(lgm, axis=1, keepdims=True)
    i1 = jnp.min(jnp.where(lgm == v1, iota, E), axis=1, keepdims=True)
    m1 = iota == i1
    ew = jnp.exp(v1 - v0)
    w0_ref[...] = 1.0 / (1.0 + ew)
    w1_ref[...] = ew / (1.0 + ew)

    a = m0.astype(jnp.float32) + m1.astype(jnp.float32)
    incl = a
    s = 1
    while s < N:
        shifted = jnp.concatenate(
            [jnp.zeros((s, E), jnp.float32), incl[:N - s, :]], axis=0)
        incl = incl + shifted
        s *= 2
    excl = incl - a
    counts = incl[N - 1:N, :]
    pcnt = jnp.ceil(counts * (1.0 / T))
    ltri = (lax.broadcasted_iota(jnp.int32, (E, E), 0)
            < lax.broadcasted_iota(jnp.int32, (E, E), 1)).astype(jnp.float32)
    offs_blk = lax.dot_general(pcnt, ltri, (((1,), (0,)), ((), ())),
                               preferred_element_type=jnp.float32)
    offs_row = offs_blk * T

    pos0 = jnp.sum(jnp.where(m0, excl, 0.0), axis=1, keepdims=True)
    pos1 = jnp.sum(jnp.where(m1, excl, 0.0), axis=1, keepdims=True)
    off0 = jnp.sum(jnp.where(m0, offs_row, 0.0), axis=1, keepdims=True)
    off1 = jnp.sum(jnp.where(m1, offs_row, 0.0), axis=1, keepdims=True)
    inv0_ref[...] = (off0 + pos0).astype(jnp.int32)
    inv1_ref[...] = (off1 + pos1).astype(jnp.int32)

    ends = offs_blk + pcnt
    b_iota = lax.broadcasted_iota(jnp.int32, (1, NB), 1).astype(jnp.float32)
    bev = jnp.zeros((1, NB), jnp.float32)
    for e in range(E):
        bev += (b_iota >= ends[0:1, e:e + 1]).astype(jnp.float32)
    be_ref[...] = jnp.minimum(bev, E - 1).astype(jnp.int32)
    na_ref[...] = ends[0:1, E - 1:E].astype(jnp.int32)


def _router(flat, router_w, router_b):
    return pl.pallas_call(
        _router_body,
        in_specs=[
            pl.BlockSpec((N, H), lambda: (0, 0)),
            pl.BlockSpec((E, H), lambda: (0, 0)),
            pl.BlockSpec((1, E), lambda: (0, 0)),
        ],
        out_specs=[
            pl.BlockSpec((N, 1), lambda: (0, 0)),
            pl.BlockSpec((N, 1), lambda: (0, 0)),
            pl.BlockSpec((N, 1), lambda: (0, 0)),
            pl.BlockSpec((N, 1), lambda: (0, 0)),
            pl.BlockSpec((1, NB), lambda: (0, 0)),
            pl.BlockSpec((1, 1), lambda: (0, 0)),
        ],
        out_shape=[
            jax.ShapeDtypeStruct((N, 1), jnp.int32),
            jax.ShapeDtypeStruct((N, 1), jnp.int32),
            jax.ShapeDtypeStruct((N, 1), jnp.float32),
            jax.ShapeDtypeStruct((N, 1), jnp.float32),
            jax.ShapeDtypeStruct((1, NB), jnp.int32),
            jax.ShapeDtypeStruct((1, 1), jnp.int32),
        ],
    )(flat, router_w, router_b)


def _sc_dispatch_gather_body(x_hbm, inv0_hbm, inv1_hbm, xg_hbm,
                             idx_v, rows_v, sem):
    c = lax.axis_index("c")
    s = lax.axis_index("s")
    wid = c * NS + s
    base = wid * TOK_W
    pltpu.sync_copy(inv0_hbm.at[pl.ds(base, TOK_W)], idx_v.at[0])
    pltpu.sync_copy(inv1_hbm.at[pl.ds(base, TOK_W)], idx_v.at[1])
    pltpu.sync_copy(x_hbm.at[pl.ds(base, TOK_W)], rows_v)
    cp0 = pltpu.async_copy(rows_v, xg_hbm.at[idx_v.at[0]], sem)
    cp1 = pltpu.async_copy(rows_v, xg_hbm.at[idx_v.at[1]], sem)
    cp0.wait()
    cp1.wait()


def _sc_dispatch_gather(flat, inv0, inv1):
    mesh = plsc.VectorSubcoreMesh(core_axis_name="c", subcore_axis_name="s")
    f = pl.kernel(
        _sc_dispatch_gather_body,
        out_type=jax.ShapeDtypeStruct((NPAD, H), jnp.float32),
        mesh=mesh,
        scratch_types=[
            pltpu.VMEM((2, TOK_W), jnp.int32),
            pltpu.VMEM((TOK_W, H), jnp.float32),
            pltpu.SemaphoreType.DMA,
        ],
    )
    return f(flat, inv0, inv1)


def _ffn_body(be_ref, na_ref, xg_ref, w1_ref, b1_ref, w2_ref, b2_ref, y_ref):
    b = pl.program_id(0)

    @pl.when(b < na_ref[0])
    def _compute():
        x = xg_ref[...].astype(jnp.bfloat16)
        h = lax.dot_general(x, w1_ref[0], (((1,), (1,)), ((), ())),
                            preferred_element_type=jnp.float32)
        h = h + b1_ref[0]
        h = 0.5 * h * (1.0 + lax.erf(h * (1.0 / math.sqrt(2.0))))
        y_ref[...] = lax.dot_general(h.astype(jnp.bfloat16), w2_ref[0],
                                     (((1,), (1,)), ((), ())),
                                     preferred_element_type=jnp.float32) + b2_ref[0]

    @pl.when(b >= na_ref[0])
    def _idle():
        y_ref[...] = jnp.zeros((T, H), jnp.float32)


def _grouped_ffn(be, na, xg, W1, b1, W2, b2):
    grid_spec = pltpu.PrefetchScalarGridSpec(
        num_scalar_prefetch=2,
        grid=(NB,),
        in_specs=[
            pl.BlockSpec((T, H), lambda b, be, na: (b, 0)),
            pl.BlockSpec((1, F, H), lambda b, be, na: (be[b], 0, 0)),
            pl.BlockSpec((1, 1, F), lambda b, be, na: (be[b], 0, 0)),
            pl.BlockSpec((1, H, F), lambda b, be, na: (be[b], 0, 0)),
            pl.BlockSpec((1, 1, H), lambda b, be, na: (be[b], 0, 0)),
        ],
        out_specs=pl.BlockSpec((T, H), lambda b, be, na: (b, 0)),
    )
    return pl.pallas_call(
        _ffn_body,
        grid_spec=grid_spec,
        out_shape=jax.ShapeDtypeStruct((NPAD, H), jnp.float32),
    )(be, na, xg, W1.astype(jnp.bfloat16), b1.reshape(E, 1, F),
      W2.astype(jnp.bfloat16), b2.reshape(E, 1, H))


def _sc_gatherback_body(y_hbm, inv0_hbm, inv1_hbm, a_hbm, b_hbm,
                        idx_v, rows_v, sem):
    c = lax.axis_index("c")
    s = lax.axis_index("s")
    wid = c * NS + s
    base = wid * TOK_W
    pltpu.sync_copy(inv0_hbm.at[pl.ds(base, TOK_W)], idx_v)
    pltpu.async_copy(y_hbm.at[idx_v], rows_v, sem).wait()
    pltpu.sync_copy(rows_v, a_hbm.at[pl.ds(base, TOK_W)])
    pltpu.sync_copy(inv1_hbm.at[pl.ds(base, TOK_W)], idx_v)
    pltpu.async_copy(y_hbm.at[idx_v], rows_v, sem).wait()
    pltpu.sync_copy(rows_v, b_hbm.at[pl.ds(base, TOK_W)])


def _sc_gatherback(y, inv0, inv1):
    mesh = plsc.VectorSubcoreMesh(core_axis_name="c", subcore_axis_name="s")
    f = pl.kernel(
        _sc_gatherback_body,
        out_type=[jax.ShapeDtypeStruct((N, H), jnp.float32),
                  jax.ShapeDtypeStruct((N, H), jnp.float32)],
        mesh=mesh,
        scratch_types=[
            pltpu.VMEM((TOK_W,), jnp.int32),
            pltpu.VMEM((TOK_W, H), jnp.float32),
            pltpu.SemaphoreType.DMA,
        ],
    )
    return f(y, inv0, inv1)


def _combine_body(a_ref, b_ref, w0_ref, w1_ref, x_ref, lng_ref, lnb_ref,
                  out_ref):
    t = w0_ref[...] * a_ref[...] + w1_ref[...] * b_ref[...] + x_ref[...]
    mu = jnp.mean(t, axis=1, keepdims=True)
    tc = t - mu
    var = jnp.mean(tc * tc, axis=1, keepdims=True)
    out_ref[...] = tc * lax.rsqrt(var + EPS) * lng_ref[...] + lnb_ref[...]


def _combine(a, b, w0, w1, flat, ln_g, ln_b):
    return pl.pallas_call(
        _combine_body,
        in_specs=[pl.BlockSpec((N, H), lambda: (0, 0)),
                  pl.BlockSpec((N, H), lambda: (0, 0)),
                  pl.BlockSpec((N, 1), lambda: (0, 0)),
                  pl.BlockSpec((N, 1), lambda: (0, 0)),
                  pl.BlockSpec((N, H), lambda: (0, 0)),
                  pl.BlockSpec((1, H), lambda: (0, 0)),
                  pl.BlockSpec((1, H), lambda: (0, 0))],
        out_specs=pl.BlockSpec((N, H), lambda: (0, 0)),
        out_shape=jax.ShapeDtypeStruct((N, H), jnp.float32),
    )(a, b, w0, w1, flat, ln_g, ln_b)


def kernel(hidden_states, router_w, router_b, W1, b1, W2, b2, ln_g, ln_b):
    bsz, seqlen, h = hidden_states.shape
    flat = hidden_states.reshape(-1, h)
    inv0, inv1, w0, w1, be, na = _router(flat, router_w,
                                         router_b.reshape(1, E))
    inv0_f = inv0.reshape(N)
    inv1_f = inv1.reshape(N)
    xg = _sc_dispatch_gather(flat, inv0_f, inv1_f)
    y = _grouped_ffn(be.reshape(NB), na.reshape(1), xg, W1, b1, W2, b2)
    a, b = _sc_gatherback(y, inv0_f, inv1_f)
    out = _combine(a, b, w0, w1, flat, ln_g.reshape(1, h), ln_b.reshape(1, h))
    return out.reshape(bsz, seqlen, h)


# speedup vs baseline: 1.0295x; 1.0295x over previous
import math

import jax
import jax.numpy as jnp
from jax import lax
from jax.experimental import pallas as pl
from jax.experimental.pallas import tpu as pltpu
from jax.experimental.pallas import tpu_sc as plsc

N = 2048
H = 768
F = 3072
E = 8
EPS = 1e-12
T = 256
NB = 24
NPAD = NB * T
FB = 768
NFB = F // FB
NC, NS = 2, 16
NW = NC * NS
TOK_SC = N // NS
ROW_W = NPAD // NW
CH = 64
NCH = ROW_W // CH
TOK_W = N // NW


def _router_body(x_ref, rw_ref, rb_ref,
                 inv0_ref, inv1_ref, w0_ref, w1_ref, be_ref, na_ref):
    x = x_ref[...]
    lg = lax.dot_general(x, rw_ref[...], (((1,), (1,)), ((), ())),
                         preferred_element_type=jnp.float32) + rb_ref[...]
    iota = lax.broadcasted_iota(jnp.int32, (N, E), 1)
    v0 = jnp.max(lg, axis=1, keepdims=True)
    i0 = jnp.min(jnp.where(lg == v0, iota, E), axis=1, keepdims=True)
    m0 = iota == i0
    lgm = jnp.where(m0, -jnp.inf, lg)
    v1 = jnp.max(lgm, axis=1, keepdims=True)
    i1 = jnp.min(jnp.where(lgm == v1, iota, E), axis=1, keepdims=True)
    m1 = iota == i1
    ew = jnp.exp(v1 - v0)
    w0_ref[...] = 1.0 / (1.0 + ew)
    w1_ref[...] = ew / (1.0 + ew)

    a = m0.astype(jnp.float32) + m1.astype(jnp.float32)
    incl = a
    s = 1
    while s < N:
        shifted = jnp.concatenate(
            [jnp.zeros((s, E), jnp.float32), incl[:N - s, :]], axis=0)
        incl = incl + shifted
        s *= 2
    excl = incl - a
    counts = incl[N - 1:N, :]
    pcnt = jnp.ceil(counts * (1.0 / T))
    ltri = (lax.broadcasted_iota(jnp.int32, (E, E), 0)
            < lax.broadcasted_iota(jnp.int32, (E, E), 1)).astype(jnp.float32)
    offs_blk = lax.dot_general(pcnt, ltri, (((1,), (0,)), ((), ())),
                               preferred_element_type=jnp.float32)
    offs_row = offs_blk * T

    pos0 = jnp.sum(jnp.where(m0, excl, 0.0), axis=1, keepdims=True)
    pos1 = jnp.sum(jnp.where(m1, excl, 0.0), axis=1, keepdims=True)
    off0 = jnp.sum(jnp.where(m0, offs_row, 0.0), axis=1, keepdims=True)
    off1 = jnp.sum(jnp.where(m1, offs_row, 0.0), axis=1, keepdims=True)
    inv0_ref[...] = (off0 + pos0).astype(jnp.int32)
    inv1_ref[...] = (off1 + pos1).astype(jnp.int32)

    ends = offs_blk + pcnt
    b_iota = lax.broadcasted_iota(jnp.int32, (1, NB), 1).astype(jnp.float32)
    bev = jnp.zeros((1, NB), jnp.float32)
    for e in range(E):
        bev += (b_iota >= ends[0:1, e:e + 1]).astype(jnp.float32)
    be_ref[...] = jnp.minimum(bev, E - 1).astype(jnp.int32)
    na_ref[...] = ends[0:1, E - 1:E].astype(jnp.int32)


def _router(flat, router_w, router_b):
    return pl.pallas_call(
        _router_body,
        in_specs=[
            pl.BlockSpec((N, H), lambda: (0, 0)),
            pl.BlockSpec((E, H), lambda: (0, 0)),
            pl.BlockSpec((1, E), lambda: (0, 0)),
        ],
        out_specs=[
            pl.BlockSpec((N, 1), lambda: (0, 0)),
            pl.BlockSpec((N, 1), lambda: (0, 0)),
            pl.BlockSpec((N, 1), lambda: (0, 0)),
            pl.BlockSpec((N, 1), lambda: (0, 0)),
            pl.BlockSpec((1, NB), lambda: (0, 0)),
            pl.BlockSpec((1, 1), lambda: (0, 0)),
        ],
        out_shape=[
            jax.ShapeDtypeStruct((N, 1), jnp.int32),
            jax.ShapeDtypeStruct((N, 1), jnp.int32),
            jax.ShapeDtypeStruct((N, 1), jnp.float32),
            jax.ShapeDtypeStruct((N, 1), jnp.float32),
            jax.ShapeDtypeStruct((1, NB), jnp.int32),
            jax.ShapeDtypeStruct((1, 1), jnp.int32),
        ],
    )(flat, router_w, router_b)


def _sc_dispatch_gather_body(x_hbm, inv0_hbm, inv1_hbm, w0_hbm, w1_hbm,
                             xg_hbm, wr_hbm, idx_v, wv_v, rows_v, sem, semw):
    c = lax.axis_index("c")
    s = lax.axis_index("s")
    wid = c * NS + s
    base = wid * TOK_W
    pltpu.sync_copy(inv0_hbm.at[pl.ds(base, TOK_W)], idx_v.at[0])
    pltpu.sync_copy(inv1_hbm.at[pl.ds(base, TOK_W)], idx_v.at[1])
    pltpu.sync_copy(w0_hbm.at[pl.ds(base, TOK_W)], wv_v.at[0])
    pltpu.sync_copy(w1_hbm.at[pl.ds(base, TOK_W)], wv_v.at[1])
    pltpu.sync_copy(x_hbm.at[pl.ds(base, TOK_W)], rows_v)
    cp0 = pltpu.async_copy(rows_v, xg_hbm.at[idx_v.at[0]], sem)
    cp1 = pltpu.async_copy(rows_v, xg_hbm.at[idx_v.at[1]], sem)
    cw0 = pltpu.async_copy(wv_v.at[0], wr_hbm.at[idx_v.at[0]], semw)
    cw1 = pltpu.async_copy(wv_v.at[1], wr_hbm.at[idx_v.at[1]], semw)
    cp0.wait()
    cp1.wait()
    cw0.wait()
    cw1.wait()


def _sc_dispatch_gather(flat, inv0, inv1, w0, w1):
    mesh = plsc.VectorSubcoreMesh(core_axis_name="c", subcore_axis_name="s")
    f = pl.kernel(
        _sc_dispatch_gather_body,
        out_type=[jax.ShapeDtypeStruct((NPAD, H), jnp.float32),
                  jax.ShapeDtypeStruct((NPAD,), jnp.float32)],
        mesh=mesh,
        scratch_types=[
            pltpu.VMEM((2, TOK_W), jnp.int32),
            pltpu.VMEM((2, TOK_W), jnp.float32),
            pltpu.VMEM((TOK_W, H), jnp.float32),
            pltpu.SemaphoreType.DMA,
            pltpu.SemaphoreType.DMA,
        ],
    )
    return f(flat, inv0, inv1, w0, w1)


def _ffn_body(be_ref, na_ref, xg_ref, w1_ref, b1_ref, w2_ref, b2_ref, wr_ref,
              y_ref):
    b = pl.program_id(0)

    @pl.when(b < na_ref[0])
    def _compute():
        x = xg_ref[...]
        h = lax.dot_general(x, w1_ref[0], (((1,), (1,)), ((), ())),
                            preferred_element_type=jnp.float32)
        h = h + b1_ref[0]
        h = 0.5 * h * (1.0 + lax.erf(h * (1.0 / math.sqrt(2.0))))
        y = lax.dot_general(h, w2_ref[0], (((1,), (1,)), ((), ())),
                            preferred_element_type=jnp.float32) + b2_ref[0]
        y_ref[...] = y * wr_ref[...]

def _grouped_ffn(be, na, xg, wr, W1, b1, W2, b2):
    grid_spec = pltpu.PrefetchScalarGridSpec(
        num_scalar_prefetch=2,
        grid=(NB,),
        in_specs=[
            pl.BlockSpec((T, H), lambda b, be, na: (b, 0)),
            pl.BlockSpec((1, F, H), lambda b, be, na: (be[b], 0, 0)),
            pl.BlockSpec((1, 1, F), lambda b, be, na: (be[b], 0, 0)),
            pl.BlockSpec((1, H, F), lambda b, be, na: (be[b], 0, 0)),
            pl.BlockSpec((1, 1, H), lambda b, be, na: (be[b], 0, 0)),
            pl.BlockSpec((T, 1), lambda b, be, na: (b, 0)),
        ],
        out_specs=pl.BlockSpec((T, H), lambda b, be, na: (b, 0)),
    )
    return pl.pallas_call(
        _ffn_body,
        grid_spec=grid_spec,
        out_shape=jax.ShapeDtypeStruct((NPAD, H), jnp.float32),
    )(be, na, xg, W1, b1.reshape(E, 1, F), W2, b2.reshape(E, 1, H),
      wr.reshape(NPAD, 1))


def _sc_gatherback_body(y_hbm, inv0_hbm, inv1_hbm, a_hbm, b_hbm,
                        idx_v, rows0_v, rows1_v, sem0, sem1):
    c = lax.axis_index("c")
    s = lax.axis_index("s")
    wid = c * NS + s
    base = wid * TOK_W
    pltpu.sync_copy(inv0_hbm.at[pl.ds(base, TOK_W)], idx_v.at[0])
    pltpu.sync_copy(inv1_hbm.at[pl.ds(base, TOK_W)], idx_v.at[1])
    cp0 = pltpu.async_copy(y_hbm.at[idx_v.at[0]], rows0_v, sem0)
    cp1 = pltpu.async_copy(y_hbm.at[idx_v.at[1]], rows1_v, sem1)
    cp0.wait()
    cw0 = pltpu.async_copy(rows0_v, a_hbm.at[pl.ds(base, TOK_W)], sem0)
    cp1.wait()
    cw1 = pltpu.async_copy(rows1_v, b_hbm.at[pl.ds(base, TOK_W)], sem1)
    cw0.wait()
    cw1.wait()


def _sc_gatherback(y, inv0, inv1):
    mesh = plsc.VectorSubcoreMesh(core_axis_name="c", subcore_axis_name="s")
    f = pl.kernel(
        _sc_gatherback_body,
        out_type=[jax.ShapeDtypeStruct((N, H), jnp.float32),
                  jax.ShapeDtypeStruct((N, H), jnp.float32)],
        mesh=mesh,
        scratch_types=[
            pltpu.VMEM((2, TOK_W), jnp.int32),
            pltpu.VMEM((TOK_W, H), jnp.float32),
            pltpu.VMEM((TOK_W, H), jnp.float32),
            pltpu.SemaphoreType.DMA,
            pltpu.SemaphoreType.DMA,
        ],
    )
    return f(y, inv0, inv1)


def _combine_body(a_ref, b_ref, x_ref, lng_ref, lnb_ref, out_ref):
    t = a_ref[...] + b_ref[...] + x_ref[...]
    mu = jnp.mean(t, axis=1, keepdims=True)
    tc = t - mu
    var = jnp.mean(tc * tc, axis=1, keepdims=True)
    out_ref[...] = tc * lax.rsqrt(var + EPS) * lng_ref[...] + lnb_ref[...]


def _combine(a, b, flat, ln_g, ln_b):
    return pl.pallas_call(
        _combine_body,
        in_specs=[pl.BlockSpec((N, H), lambda: (0, 0)),
                  pl.BlockSpec((N, H), lambda: (0, 0)),
                  pl.BlockSpec((N, H), lambda: (0, 0)),
                  pl.BlockSpec((1, H), lambda: (0, 0)),
                  pl.BlockSpec((1, H), lambda: (0, 0))],
        out_specs=pl.BlockSpec((N, H), lambda: (0, 0)),
        out_shape=jax.ShapeDtypeStruct((N, H), jnp.float32),
    )(a, b, flat, ln_g, ln_b)


def kernel(hidden_states, router_w, router_b, W1, b1, W2, b2, ln_g, ln_b):
    bsz, seqlen, h = hidden_states.shape
    flat = hidden_states.reshape(-1, h)
    inv0, inv1, w0, w1, be, na = _router(flat, router_w,
                                         router_b.reshape(1, E))
    inv0_f = inv0.reshape(N)
    inv1_f = inv1.reshape(N)
    xg, wr = _sc_dispatch_gather(flat, inv0_f, inv1_f,
                                 w0.reshape(N), w1.reshape(N))
    y = _grouped_ffn(be.reshape(NB), na.reshape(1), xg, wr, W1, b1, W2, b2)
    a, b = _sc_gatherback(y, inv0_f, inv1_f)
    out = _combine(a, b, flat, ln_g.reshape(1, h), ln_b.reshape(1, h))
    return out.reshape(bsz, seqlen, h)


# speedup vs baseline: 1.2899x; 1.2530x over previous
import math

import jax
import jax.numpy as jnp
from jax import lax
from jax.experimental import pallas as pl
from jax.experimental.pallas import tpu as pltpu
from jax.experimental.pallas import tpu_sc as plsc

N = 2048
H = 768
F = 3072
E = 8
EPS = 1e-12
T = 256
NB = 24
NPAD = NB * T
FB = 768
NFB = F // FB
NC, NS = 2, 16
NW = NC * NS
TOK_SC = N // NS
ROW_W = NPAD // NW
CH = 64
NCH = ROW_W // CH
TOK_W = N // NW


def _router_body(x_ref, rw_ref, rb_ref,
                 inv0_ref, inv1_ref, w0_ref, w1_ref, be_ref, na_ref):
    x = x_ref[...]
    lg = lax.dot_general(x, rw_ref[...], (((1,), (1,)), ((), ())),
                         preferred_element_type=jnp.float32) + rb_ref[...]
    iota = lax.broadcasted_iota(jnp.int32, (N, E), 1)
    v0 = jnp.max(lg, axis=1, keepdims=True)
    i0 = jnp.min(jnp.where(lg == v0, iota, E), axis=1, keepdims=True)
    m0 = iota == i0
    lgm = jnp.where(m0, -jnp.inf, lg)
    v1 = jnp.max(lgm, axis=1, keepdims=True)
    i1 = jnp.min(jnp.where(lgm == v1, iota, E), axis=1, keepdims=True)
    m1 = iota == i1
    ew = jnp.exp(v1 - v0)
    w0_ref[...] = 1.0 / (1.0 + ew)
    w1_ref[...] = ew / (1.0 + ew)

    a = m0.astype(jnp.float32) + m1.astype(jnp.float32)
    incl = a
    s = 1
    while s < N:
        shifted = jnp.concatenate(
            [jnp.zeros((s, E), jnp.float32), incl[:N - s, :]], axis=0)
        incl = incl + shifted
        s *= 2
    excl = incl - a
    counts = incl[N - 1:N, :]
    pcnt = jnp.ceil(counts * (1.0 / T))
    ltri = (lax.broadcasted_iota(jnp.int32, (E, E), 0)
            < lax.broadcasted_iota(jnp.int32, (E, E), 1)).astype(jnp.float32)
    offs_blk = lax.dot_general(pcnt, ltri, (((1,), (0,)), ((), ())),
                               preferred_element_type=jnp.float32)
    offs_row = offs_blk * T

    pos0 = jnp.sum(jnp.where(m0, excl, 0.0), axis=1, keepdims=True)
    pos1 = jnp.sum(jnp.where(m1, excl, 0.0), axis=1, keepdims=True)
    off0 = jnp.sum(jnp.where(m0, offs_row, 0.0), axis=1, keepdims=True)
    off1 = jnp.sum(jnp.where(m1, offs_row, 0.0), axis=1, keepdims=True)
    inv0_ref[...] = (off0 + pos0).astype(jnp.int32)
    inv1_ref[...] = (off1 + pos1).astype(jnp.int32)

    ends = offs_blk + pcnt
    b_iota = lax.broadcasted_iota(jnp.int32, (1, NB), 1).astype(jnp.float32)
    bev = jnp.zeros((1, NB), jnp.float32)
    for e in range(E):
        bev += (b_iota >= ends[0:1, e:e + 1]).astype(jnp.float32)
    be_ref[...] = jnp.minimum(bev, E - 1).astype(jnp.int32)
    na_ref[...] = ends[0:1, E - 1:E].astype(jnp.int32)


def _router(flat, router_w, router_b):
    return pl.pallas_call(
        _router_body,
        in_specs=[
            pl.BlockSpec((N, H), lambda: (0, 0)),
            pl.BlockSpec((E, H), lambda: (0, 0)),
            pl.BlockSpec((1, E), lambda: (0, 0)),
        ],
        out_specs=[
            pl.BlockSpec((N, 1), lambda: (0, 0)),
            pl.BlockSpec((N, 1), lambda: (0, 0)),
            pl.BlockSpec((N, 1), lambda: (0, 0)),
            pl.BlockSpec((N, 1), lambda: (0, 0)),
            pl.BlockSpec((1, NB), lambda: (0, 0)),
            pl.BlockSpec((1, 1), lambda: (0, 0)),
        ],
        out_shape=[
            jax.ShapeDtypeStruct((N, 1), jnp.int32),
            jax.ShapeDtypeStruct((N, 1), jnp.int32),
            jax.ShapeDtypeStruct((N, 1), jnp.float32),
            jax.ShapeDtypeStruct((N, 1), jnp.float32),
            jax.ShapeDtypeStruct((1, NB), jnp.int32),
            jax.ShapeDtypeStruct((1, 1), jnp.int32),
        ],
    )(flat, router_w, router_b)


def _sc_dispatch_gather_body(x_hbm, inv0_hbm, inv1_hbm, xg_hbm,
                             idx_v, rows_v, sem):
    c = lax.axis_index("c")
    s = lax.axis_index("s")
    wid = c * NS + s
    base = wid * TOK_W
    pltpu.sync_copy(inv0_hbm.at[pl.ds(base, TOK_W)], idx_v.at[0])
    pltpu.sync_copy(inv1_hbm.at[pl.ds(base, TOK_W)], idx_v.at[1])
    pltpu.sync_copy(x_hbm.at[pl.ds(base, TOK_W)], rows_v)
    cp0 = pltpu.async_copy(rows_v, xg_hbm.at[idx_v.at[0]], sem)
    cp1 = pltpu.async_copy(rows_v, xg_hbm.at[idx_v.at[1]], sem)
    cp0.wait()
    cp1.wait()


def _sc_dispatch_gather(flat, inv0, inv1):
    mesh = plsc.VectorSubcoreMesh(core_axis_name="c", subcore_axis_name="s")
    f = pl.kernel(
        _sc_dispatch_gather_body,
        out_type=jax.ShapeDtypeStruct((NPAD, H), jnp.float32),
        mesh=mesh,
        scratch_types=[
            pltpu.VMEM((2, TOK_W), jnp.int32),
            pltpu.VMEM((TOK_W, H), jnp.float32),
            pltpu.SemaphoreType.DMA,
        ],
    )
    return f(flat, inv0, inv1)


def _ffn_body(be_ref, na_ref, xg_ref, w1_ref, b1_ref, w2_ref, b2_ref, y_ref):
    b = pl.program_id(0)

    @pl.when(b < na_ref[0])
    def _compute():
        x = xg_ref[...]
        h = lax.dot_general(x, w1_ref[0], (((1,), (1,)), ((), ())),
                            preferred_element_type=jnp.float32)
        h = h + b1_ref[0]
        h = 0.5 * h * (1.0 + lax.erf(h * (1.0 / math.sqrt(2.0))))
        y_ref[...] = lax.dot_general(h, w2_ref[0], (((1,), (1,)), ((), ())),
                                     preferred_element_type=jnp.float32) + b2_ref[0]

    @pl.when(b >= na_ref[0])
    def _idle():
        y_ref[...] = jnp.zeros((T, H), jnp.float32)


def _grouped_ffn(be, na, xg, W1, b1, W2, b2):
    grid_spec = pltpu.PrefetchScalarGridSpec(
        num_scalar_prefetch=2,
        grid=(NB,),
        in_specs=[
            pl.BlockSpec((T, H), lambda b, be, na: (b, 0)),
            pl.BlockSpec((1, F, H), lambda b, be, na: (be[b], 0, 0)),
            pl.BlockSpec((1, 1, F), lambda b, be, na: (be[b], 0, 0)),
            pl.BlockSpec((1, H, F), lambda b, be, na: (be[b], 0, 0)),
            pl.BlockSpec((1, 1, H), lambda b, be, na: (be[b], 0, 0)),
        ],
        out_specs=pl.BlockSpec((T, H), lambda b, be, na: (b, 0)),
    )
    return pl.pallas_call(
        _ffn_body,
        grid_spec=grid_spec,
        out_shape=jax.ShapeDtypeStruct((NPAD, H), jnp.float32),
    )(be, na, xg, W1, b1.reshape(E, 1, F), W2, b2.reshape(E, 1, H))


def _sc_gatherback_body(y_hbm, inv0_hbm, inv1_hbm, a_hbm, b_hbm,
                        idx_v, rows0_v, rows1_v, sem0, sem1):
    c = lax.axis_index("c")
    s = lax.axis_index("s")
    wid = c * NS + s
    base = wid * TOK_W
    pltpu.sync_copy(inv0_hbm.at[pl.ds(base, TOK_W)], idx_v.at[0])
    pltpu.sync_copy(inv1_hbm.at[pl.ds(base, TOK_W)], idx_v.at[1])
    cp0 = pltpu.async_copy(y_hbm.at[idx_v.at[0]], rows0_v, sem0)
    cp1 = pltpu.async_copy(y_hbm.at[idx_v.at[1]], rows1_v, sem1)
    cp0.wait()
    cw0 = pltpu.async_copy(rows0_v, a_hbm.at[pl.ds(base, TOK_W)], sem0)
    cp1.wait()
    cw1 = pltpu.async_copy(rows1_v, b_hbm.at[pl.ds(base, TOK_W)], sem1)
    cw0.wait()
    cw1.wait()


def _sc_gatherback(y, inv0, inv1):
    mesh = plsc.VectorSubcoreMesh(core_axis_name="c", subcore_axis_name="s")
    f = pl.kernel(
        _sc_gatherback_body,
        out_type=[jax.ShapeDtypeStruct((N, H), jnp.float32),
                  jax.ShapeDtypeStruct((N, H), jnp.float32)],
        mesh=mesh,
        scratch_types=[
            pltpu.VMEM((2, TOK_W), jnp.int32),
            pltpu.VMEM((TOK_W, H), jnp.float32),
            pltpu.VMEM((TOK_W, H), jnp.float32),
            pltpu.SemaphoreType.DMA,
            pltpu.SemaphoreType.DMA,
        ],
    )
    return f(y, inv0, inv1)


def _combine_body(a_ref, b_ref, w0_ref, w1_ref, x_ref, lng_ref, lnb_ref,
                  out_ref):
    t = w0_ref[...] * a_ref[...] + w1_ref[...] * b_ref[...] + x_ref[...]
    mu = jnp.mean(t, axis=1, keepdims=True)
    tc = t - mu
    var = jnp.mean(tc * tc, axis=1, keepdims=True)
    out_ref[...] = tc * lax.rsqrt(var + EPS) * lng_ref[...] + lnb_ref[...]


def _combine(a, b, w0, w1, flat, ln_g, ln_b):
    return pl.pallas_call(
        _combine_body,
        in_specs=[pl.BlockSpec((N, H), lambda: (0, 0)),
                  pl.BlockSpec((N, H), lambda: (0, 0)),
                  pl.BlockSpec((N, 1), lambda: (0, 0)),
                  pl.BlockSpec((N, 1), lambda: (0, 0)),
                  pl.BlockSpec((N, H), lambda: (0, 0)),
                  pl.BlockSpec((1, H), lambda: (0, 0)),
                  pl.BlockSpec((1, H), lambda: (0, 0))],
        out_specs=pl.BlockSpec((N, H), lambda: (0, 0)),
        out_shape=jax.ShapeDtypeStruct((N, H), jnp.float32),
    )(a, b, w0, w1, flat, ln_g, ln_b)


def kernel(hidden_states, router_w, router_b, W1, b1, W2, b2, ln_g, ln_b):
    bsz, seqlen, h = hidden_states.shape
    flat = hidden_states.reshape(-1, h)
    inv0, inv1, w0, w1, be, na = _router(flat, router_w,
                                         router_b.reshape(1, E))
    inv0_f = inv0.reshape(N)
    inv1_f = inv1.reshape(N)
    xg = _sc_dispatch_gather(flat, inv0_f, inv1_f)
    y = _grouped_ffn(be.reshape(NB), na.reshape(1), xg, W1, b1, W2, b2)
    a, b = _sc_gatherback(y, inv0_f, inv1_f)
    out = _combine(a, b, w0, w1, flat, ln_g.reshape(1, h), ln_b.reshape(1, h))
    return out.reshape(bsz, seqlen, h)
